# baseline (speedup 1.0000x reference)
import functools

import jax
import jax.numpy as jnp
from jax import lax
from jax.experimental import pallas as pl
from jax.experimental.pallas import tpu as pltpu
from jax.experimental.pallas import tpu_sc as plsc

N = 10000
NP = 10240
E = 320000
D = 128
NC = 2
NS = 16
NW = NC * NS
IW = 128
KPT = 80
SCH = 64
NSUB = KPT * IW // SCH
EP = NW * KPT * IW
RPT = NP // NS
BT = 1024
NG = NP // BT


def _sc_mesh():
    return plsc.VectorSubcoreMesh(core_axis_name="c", subcore_axis_name="s")


def _copy_idx_row(idx_all, c, buf):
    def cp(j, _):
        buf[pl.ds(j * 16, 16)] = idx_all[c, pl.ds(j * 16, 16)]
        return 0

    lax.fori_loop(0, IW // 16, cp, 0)


def _copy_idx_sub(idx_all, u, buf):
    row = u // 2
    half = u % 2

    def cp(j, _):
        buf[pl.ds(j * 16, 16)] = idx_all[row, pl.ds(half * SCH + j * 16, 16)]
        return 0

    lax.fori_loop(0, SCH // 16, cp, 0)


def _sc_degrees(src2d, dst2d):
    @functools.partial(
        pl.kernel,
        out_type=jax.ShapeDtypeStruct((NC, 2, NP), jnp.float32),
        mesh=_sc_mesh(),
        scratch_types=[
            pltpu.VMEM((KPT, IW), jnp.int32),
            pltpu.VMEM((KPT, IW), jnp.int32),
            pltpu.VMEM((IW,), jnp.int32),
            pltpu.VMEM((IW,), jnp.int32),
            pltpu.VMEM((IW,), jnp.int32),
            pltpu.VMEM((IW,), jnp.int32),
            pltpu.VMEM((IW,), jnp.float32),
            pltpu.VMEM((RPT,), jnp.float32),
            pltpu.VMEM_SHARED((NP,), jnp.float32),
            pltpu.VMEM_SHARED((NP,), jnp.float32),
            pltpu.SemaphoreType.DMA,
            pltpu.SemaphoreType.DMA,
        ],
    )
    def deg_kernel(src_hbm, dst_hbm, out_hbm, sidx, didx, sa, da, sb, db,
                   ones_v, zero_v, acc_s, acc_d, sem_s, sem_d):
        cid = lax.axis_index("c")
        sid = lax.axis_index("s")
        w = cid * NS + sid

        def fill_ones(i, _):
            ones_v[pl.ds(i * 16, 16)] = jnp.ones((16,), jnp.float32)
            return 0

        lax.fori_loop(0, IW // 16, fill_ones, 0)

        def fill_zero(i, _):
            zero_v[pl.ds(i * 16, 16)] = jnp.zeros((16,), jnp.float32)
            return 0

        lax.fori_loop(0, RPT // 16, fill_zero, 0)

        pltpu.sync_copy(zero_v, acc_s.at[pl.ds(sid * RPT, RPT)])
        pltpu.sync_copy(zero_v, acc_d.at[pl.ds(sid * RPT, RPT)])
        pltpu.sync_copy(src_hbm.at[pl.ds(w * KPT, KPT)], sidx)
        pltpu.sync_copy(dst_hbm.at[pl.ds(w * KPT, KPT)], didx)
        plsc.subcore_barrier()

        _copy_idx_row(sidx, 0, sa)
        _copy_idx_row(didx, 0, da)

        def step(c2, _):
            c = 2 * c2
            cs = pltpu.async_copy(ones_v, acc_s.at[sa], sem_s, add=True)
            cd = pltpu.async_copy(ones_v, acc_d.at[da], sem_d, add=True)
            _copy_idx_row(sidx, c + 1, sb)
            _copy_idx_row(didx, c + 1, db)
            cs.wait()
            cd.wait()
            cs = pltpu.async_copy(ones_v, acc_s.at[sb], sem_s, add=True)
            cd = pltpu.async_copy(ones_v, acc_d.at[db], sem_d, add=True)

            @pl.when(c + 2 < KPT)
            def _():
                _copy_idx_row(sidx, c + 2, sa)
                _copy_idx_row(didx, c + 2, da)

            cs.wait()
            cd.wait()
            return 0

        lax.fori_loop(0, KPT // 2, step, 0)
        plsc.subcore_barrier()

        r0 = sid * RPT
        pltpu.sync_copy(acc_s.at[pl.ds(r0, RPT)], out_hbm.at[cid, 0, pl.ds(r0, RPT)])
        pltpu.sync_copy(acc_d.at[pl.ds(r0, RPT)], out_hbm.at[cid, 1, pl.ds(r0, RPT)])

    return deg_kernel(src2d, dst2d)


def _sc_scatter(hw, src2d, dst2d):
    @functools.partial(
        pl.kernel,
        out_type=jax.ShapeDtypeStruct((NC, NP, D), jnp.float32),
        mesh=_sc_mesh(),
        scratch_types=[
            pltpu.VMEM((KPT, IW), jnp.int32),
            pltpu.VMEM((KPT, IW), jnp.int32),
            pltpu.VMEM((IW,), jnp.int32),
            pltpu.VMEM((IW,), jnp.int32),
            pltpu.VMEM((IW, D), jnp.float32),
            pltpu.VMEM_SHARED((NP, D), jnp.float32),
            pltpu.SemaphoreType.DMA,
        ],
    )
    def msg_kernel(hw_hbm, src_hbm, dst_hbm, out_hbm, sidx, didx, sa, da,
                   rows_a, acc, sem_a):
        cid = lax.axis_index("c")
        sid = lax.axis_index("s")
        w = cid * NS + sid

        def zrow(i, _):
            def zcol(j, _):
                rows_a[i, pl.ds(j * 16, 16)] = jnp.zeros((16,), jnp.float32)
                return 0

            return lax.fori_loop(0, D // 16, zcol, 0)

        lax.fori_loop(0, IW, zrow, 0)

        def zcopy(k, _):
            pltpu.sync_copy(rows_a, acc.at[pl.ds(sid * RPT + k * IW, IW)])
            return 0

        lax.fori_loop(0, RPT // IW, zcopy, 0)

        pltpu.sync_copy(src_hbm.at[pl.ds(w * KPT, KPT)], sidx)
        pltpu.sync_copy(dst_hbm.at[pl.ds(w * KPT, KPT)], didx)
        plsc.subcore_barrier()

        def step(c, _):
            _copy_idx_row(sidx, c, sa)
            _copy_idx_row(didx, c, da)
            pltpu.async_copy(hw_hbm.at[sa], rows_a, sem_a).wait()
            pltpu.sync_copy(rows_a, acc.at[da], add=True)
            return 0

        lax.fori_loop(0, KPT, step, 0)
        plsc.subcore_barrier()

        r0 = sid * RPT
        pltpu.sync_copy(acc.at[pl.ds(r0, RPT)], out_hbm.at[cid, pl.ds(r0, RPT)])

    return msg_kernel(hw, src2d, dst2d)


def _norm(degp):
    d = degp[0] + degp[1]
    return jnp.where(d > 0, 1.0 / jnp.sqrt(jnp.maximum(d, 1.0)), 0.0)


def _tc_first(xp, degoutp, W1):
    def body(x_ref, dop_ref, w_ref, o_ref):
        ns = _norm(dop_ref[...])
        o_ref[...] = jnp.dot(x_ref[...] * ns, w_ref[...],
                             preferred_element_type=jnp.float32)

    return pl.pallas_call(
        body,
        grid=(NG,),
        in_specs=[
            pl.BlockSpec((BT, D), lambda i: (i, 0)),
            pl.BlockSpec((NC, BT, 1), lambda i: (0, i, 0)),
            pl.BlockSpec((D, D), lambda i: (0, 0)),
        ],
        out_specs=pl.BlockSpec((BT, D), lambda i: (i, 0)),
        out_shape=jax.ShapeDtypeStruct((NP, D), jnp.float32),
    )(xp, degoutp, W1)


def _tc_mid(aggp, deginp, degoutp, b, W):
    def body(a_ref, dip_ref, dop_ref, b_ref, w_ref, o_ref):
        a = a_ref[0] + a_ref[1]
        nd = _norm(dip_ref[...])
        h = jnp.maximum(a * nd + b_ref[...][None, :], 0.0)
        ns = _norm(dop_ref[...])
        o_ref[...] = jnp.dot(h * ns, w_ref[...],
                             preferred_element_type=jnp.float32)

    return pl.pallas_call(
        body,
        grid=(NG,),
        in_specs=[
            pl.BlockSpec((NC, BT, D), lambda i: (0, i, 0)),
            pl.BlockSpec((NC, BT, 1), lambda i: (0, i, 0)),
            pl.BlockSpec((NC, BT, 1), lambda i: (0, i, 0)),
            pl.BlockSpec((D,), lambda i: (0,)),
            pl.BlockSpec((D, D), lambda i: (0, 0)),
        ],
        out_specs=pl.BlockSpec((BT, D), lambda i: (i, 0)),
        out_shape=jax.ShapeDtypeStruct((NP, D), jnp.float32),
    )(aggp, deginp, degoutp, b, W)


def _tc_final(aggp, deginp, b3, Wl, bl):
    def body(a_ref, dip_ref, b_ref, wl_ref, bl_ref, o_ref):
        i = pl.program_id(0)
        a = a_ref[0] + a_ref[1]
        nd = _norm(dip_ref[...])
        h = jnp.maximum(a * nd + b_ref[...][None, :], 0.0)
        rid = lax.broadcasted_iota(jnp.int32, (BT, 1), 0) + i * BT
        h = jnp.where(rid < N, h, 0.0)
        wl = wl_ref[...][:, 0]
        p = jnp.sum(h * wl[None, :])

        @pl.when(i == 0)
        def _init():
            o_ref[...] = jnp.zeros_like(o_ref)

        o_ref[...] += p

        @pl.when(i == NG - 1)
        def _fin():
            o_ref[...] = o_ref[...] / float(N) + bl_ref[...][None, :]

    return pl.pallas_call(
        body,
        grid=(NG,),
        in_specs=[
            pl.BlockSpec((NC, BT, D), lambda i: (0, i, 0)),
            pl.BlockSpec((NC, BT, 1), lambda i: (0, i, 0)),
            pl.BlockSpec((D,), lambda i: (0,)),
            pl.BlockSpec((D, 1), lambda i: (0, 0)),
            pl.BlockSpec((1,), lambda i: (0,)),
        ],
        out_specs=pl.BlockSpec((1, 1), lambda i: (0, 0)),
        out_shape=jax.ShapeDtypeStruct((1, 1), jnp.float32),
    )(aggp, deginp, b3, Wl, bl)


def kernel(x, edge_index, W1, b1, W2, b2, W3, b3, Wl, bl):
    pad = jnp.full((EP - E,), N, dtype=edge_index.dtype)
    src = jnp.concatenate([edge_index[0], pad]).reshape(NW * KPT, IW)
    dst = jnp.concatenate([edge_index[1], pad]).reshape(NW * KPT, IW)
    xp = jnp.pad(x, ((0, NP - N), (0, 0)))

    degp = _sc_degrees(src, dst)
    degsrc = degp[:, 0, :].reshape(NC, NP, 1)
    degdst = degp[:, 1, :].reshape(NC, NP, 1)

    hw1 = _tc_first(xp, degsrc, W1)
    agg1 = _sc_scatter(hw1, src, dst)
    hw2 = _tc_mid(agg1, degdst, degsrc, b1, W2)
    agg2 = _sc_scatter(hw2, src, dst)
    hw3 = _tc_mid(agg2, degdst, degsrc, b2, W3)
    agg3 = _sc_scatter(hw3, src, dst)
    return _tc_final(agg3, degdst, b3, Wl, bl)


# speedup vs baseline: 3.0951x; 3.0951x over previous
import functools

import jax
import jax.numpy as jnp
from jax import lax
from jax.experimental import pallas as pl
from jax.experimental.pallas import tpu as pltpu
from jax.experimental.pallas import tpu_sc as plsc

N = 10000
NP = 10240
E = 320000
D = 128
NC = 2
NS = 16
NW = NC * NS
IW = 128
KPT = 80
SCH = 64
NSUB = KPT * IW // SCH
EP = NW * KPT * IW
RPT = NP // NS
BT = 1024
NG = NP // BT


def _sc_mesh():
    return plsc.VectorSubcoreMesh(core_axis_name="c", subcore_axis_name="s")


def _copy_idx_row(idx_all, c, buf):
    def cp(j, _):
        buf[pl.ds(j * 16, 16)] = idx_all[c, pl.ds(j * 16, 16)]
        return 0

    lax.fori_loop(0, IW // 16, cp, 0)


def _copy_idx_sub(idx_all, u, buf):
    row = u // 2
    half = u % 2

    def cp(j, _):
        buf[pl.ds(j * 16, 16)] = idx_all[row, pl.ds(half * SCH + j * 16, 16)]
        return 0

    lax.fori_loop(0, SCH // 16, cp, 0)


def _sc_degrees(src2d, dst2d):
    @functools.partial(
        pl.kernel,
        out_type=jax.ShapeDtypeStruct((NC, 2, NP), jnp.float32),
        mesh=_sc_mesh(),
        scratch_types=[
            pltpu.VMEM((KPT, IW), jnp.int32),
            pltpu.VMEM((KPT, IW), jnp.int32),
            pltpu.VMEM((IW,), jnp.int32),
            pltpu.VMEM((IW,), jnp.int32),
            pltpu.VMEM((IW,), jnp.int32),
            pltpu.VMEM((IW,), jnp.int32),
            pltpu.VMEM((IW,), jnp.float32),
            pltpu.VMEM((RPT,), jnp.float32),
            pltpu.VMEM_SHARED((NP,), jnp.float32),
            pltpu.VMEM_SHARED((NP,), jnp.float32),
            pltpu.SemaphoreType.DMA,
            pltpu.SemaphoreType.DMA,
        ],
    )
    def deg_kernel(src_hbm, dst_hbm, out_hbm, sidx, didx, sa, da, sb, db,
                   ones_v, zero_v, acc_s, acc_d, sem_s, sem_d):
        cid = lax.axis_index("c")
        sid = lax.axis_index("s")
        w = cid * NS + sid

        def fill_ones(i, _):
            ones_v[pl.ds(i * 16, 16)] = jnp.ones((16,), jnp.float32)
            return 0

        lax.fori_loop(0, IW // 16, fill_ones, 0)

        def fill_zero(i, _):
            zero_v[pl.ds(i * 16, 16)] = jnp.zeros((16,), jnp.float32)
            return 0

        lax.fori_loop(0, RPT // 16, fill_zero, 0)

        pltpu.sync_copy(zero_v, acc_s.at[pl.ds(sid * RPT, RPT)])
        pltpu.sync_copy(zero_v, acc_d.at[pl.ds(sid * RPT, RPT)])
        pltpu.sync_copy(src_hbm.at[pl.ds(w * KPT, KPT)], sidx)
        pltpu.sync_copy(dst_hbm.at[pl.ds(w * KPT, KPT)], didx)
        plsc.subcore_barrier()

        _copy_idx_row(sidx, 0, sa)
        _copy_idx_row(didx, 0, da)

        def step(c2, _):
            c = 2 * c2
            cs = pltpu.async_copy(ones_v, acc_s.at[sa], sem_s, add=True)
            cd = pltpu.async_copy(ones_v, acc_d.at[da], sem_d, add=True)
            _copy_idx_row(sidx, c + 1, sb)
            _copy_idx_row(didx, c + 1, db)
            cs.wait()
            cd.wait()
            cs = pltpu.async_copy(ones_v, acc_s.at[sb], sem_s, add=True)
            cd = pltpu.async_copy(ones_v, acc_d.at[db], sem_d, add=True)

            @pl.when(c + 2 < KPT)
            def _():
                _copy_idx_row(sidx, c + 2, sa)
                _copy_idx_row(didx, c + 2, da)

            cs.wait()
            cd.wait()
            return 0

        lax.fori_loop(0, KPT // 2, step, 0)
        plsc.subcore_barrier()

        r0 = sid * RPT
        pltpu.sync_copy(acc_s.at[pl.ds(r0, RPT)], out_hbm.at[cid, 0, pl.ds(r0, RPT)])
        pltpu.sync_copy(acc_d.at[pl.ds(r0, RPT)], out_hbm.at[cid, 1, pl.ds(r0, RPT)])

    return deg_kernel(src2d, dst2d)


CH = 80
ECT = E // NW
KCH = ECT // CH


def _sc_scatter(hw, srcf, dstf):
    @functools.partial(
        pl.kernel,
        out_type=jax.ShapeDtypeStruct((NC, NP, D), jnp.float32),
        mesh=_sc_mesh(),
        scratch_types=[
            pltpu.VMEM((CH,), jnp.int32),
            pltpu.VMEM((CH,), jnp.int32),
            pltpu.VMEM((CH,), jnp.int32),
            pltpu.VMEM((CH,), jnp.int32),
            pltpu.VMEM((CH, D), jnp.float32),
            pltpu.VMEM((CH, D), jnp.float32),
            pltpu.VMEM_SHARED((NP, D), jnp.float32),
            pltpu.SemaphoreType.DMA,
            pltpu.SemaphoreType.DMA,
            pltpu.SemaphoreType.DMA,
            pltpu.SemaphoreType.DMA,
        ],
    )
    def msg_kernel(hw_hbm, src_hbm, dst_hbm, out_hbm, si_a, di_a, si_b, di_b,
                   rows_a, rows_b, acc, sem_ga, sem_gb, sem_ia, sem_ib):
        cid = lax.axis_index("c")
        sid = lax.axis_index("s")
        w = cid * NS + sid
        base = w * ECT

        def zrow(i, _):
            def zcol(j, _):
                rows_a[i, pl.ds(j * 16, 16)] = jnp.zeros((16,), jnp.float32)
                return 0

            return lax.fori_loop(0, D // 16, zcol, 0)

        lax.fori_loop(0, CH, zrow, 0)

        def zcopy(k, _):
            pltpu.sync_copy(rows_a, acc.at[pl.ds(sid * RPT + k * CH, CH)])
            return 0

        lax.fori_loop(0, RPT // CH, zcopy, 0)
        plsc.subcore_barrier()

        def fire_idx(c, si, di, sem):
            off = base + c * CH
            pltpu.async_copy(src_hbm.at[pl.ds(off, CH)], si, sem)
            pltpu.async_copy(dst_hbm.at[pl.ds(off, CH)], di, sem)

        def wait_idx(si, di, sem):
            pltpu.make_async_copy(src_hbm.at[pl.ds(0, CH)], si, sem).wait()
            pltpu.make_async_copy(dst_hbm.at[pl.ds(0, CH)], di, sem).wait()

        def wait_gather(rows, sem):
            pltpu.make_async_copy(hw_hbm.at[pl.ds(0, CH)], rows, sem).wait()

        fire_idx(0, si_a, di_a, sem_ia)
        fire_idx(1, si_b, di_b, sem_ib)
        wait_idx(si_a, di_a, sem_ia)
        pltpu.async_copy(hw_hbm.at[si_a], rows_a, sem_ga)

        def step(c2, _):
            c = 2 * c2
            wait_idx(si_b, di_b, sem_ib)
            wait_gather(rows_a, sem_ga)
            pltpu.async_copy(hw_hbm.at[si_b], rows_b, sem_gb)
            pltpu.sync_copy(rows_a, acc.at[di_a], add=True)
            fire_idx(c + 2, si_a, di_a, sem_ia)
            wait_idx(si_a, di_a, sem_ia)
            wait_gather(rows_b, sem_gb)
            pltpu.async_copy(hw_hbm.at[si_a], rows_a, sem_ga)
            pltpu.sync_copy(rows_b, acc.at[di_b], add=True)

            @pl.when(c + 3 < KCH)
            def _():
                fire_idx(c + 3, si_b, di_b, sem_ib)

            return 0

        lax.fori_loop(0, KCH // 2, step, 0)
        wait_gather(rows_a, sem_ga)
        pltpu.sync_copy(rows_a, acc.at[di_a], add=True)
        plsc.subcore_barrier()

        r0 = sid * RPT
        pltpu.sync_copy(acc.at[pl.ds(r0, RPT)], out_hbm.at[cid, pl.ds(r0, RPT)])

    return msg_kernel(hw, srcf, dstf)


def _norm(degp):
    d = degp[0] + degp[1]
    return jnp.where(d > 0, 1.0 / jnp.sqrt(jnp.maximum(d, 1.0)), 0.0)


def _tc_first(xp, degoutp, W1):
    def body(x_ref, dop_ref, w_ref, o_ref):
        ns = _norm(dop_ref[...])
        o_ref[...] = jnp.dot(x_ref[...] * ns, w_ref[...],
                             preferred_element_type=jnp.float32)

    return pl.pallas_call(
        body,
        grid=(NG,),
        in_specs=[
            pl.BlockSpec((BT, D), lambda i: (i, 0)),
            pl.BlockSpec((NC, BT, 1), lambda i: (0, i, 0)),
            pl.BlockSpec((D, D), lambda i: (0, 0)),
        ],
        out_specs=pl.BlockSpec((BT, D), lambda i: (i, 0)),
        out_shape=jax.ShapeDtypeStruct((NP, D), jnp.float32),
    )(xp, degoutp, W1)


def _tc_mid(aggp, deginp, degoutp, b, W):
    def body(a_ref, dip_ref, dop_ref, b_ref, w_ref, o_ref):
        a = a_ref[0] + a_ref[1]
        nd = _norm(dip_ref[...])
        h = jnp.maximum(a * nd + b_ref[...][None, :], 0.0)
        ns = _norm(dop_ref[...])
        o_ref[...] = jnp.dot(h * ns, w_ref[...],
                             preferred_element_type=jnp.float32)

    return pl.pallas_call(
        body,
        grid=(NG,),
        in_specs=[
            pl.BlockSpec((NC, BT, D), lambda i: (0, i, 0)),
            pl.BlockSpec((NC, BT, 1), lambda i: (0, i, 0)),
            pl.BlockSpec((NC, BT, 1), lambda i: (0, i, 0)),
            pl.BlockSpec((D,), lambda i: (0,)),
            pl.BlockSpec((D, D), lambda i: (0, 0)),
        ],
        out_specs=pl.BlockSpec((BT, D), lambda i: (i, 0)),
        out_shape=jax.ShapeDtypeStruct((NP, D), jnp.float32),
    )(aggp, deginp, degoutp, b, W)


def _tc_final(aggp, deginp, b3, Wl, bl):
    def body(a_ref, dip_ref, b_ref, wl_ref, bl_ref, o_ref):
        i = pl.program_id(0)
        a = a_ref[0] + a_ref[1]
        nd = _norm(dip_ref[...])
        h = jnp.maximum(a * nd + b_ref[...][None, :], 0.0)
        rid = lax.broadcasted_iota(jnp.int32, (BT, 1), 0) + i * BT
        h = jnp.where(rid < N, h, 0.0)
        wl = wl_ref[...][:, 0]
        p = jnp.sum(h * wl[None, :])

        @pl.when(i == 0)
        def _init():
            o_ref[...] = jnp.zeros_like(o_ref)

        o_ref[...] += p

        @pl.when(i == NG - 1)
        def _fin():
            o_ref[...] = o_ref[...] / float(N) + bl_ref[...][None, :]

    return pl.pallas_call(
        body,
        grid=(NG,),
        in_specs=[
            pl.BlockSpec((NC, BT, D), lambda i: (0, i, 0)),
            pl.BlockSpec((NC, BT, 1), lambda i: (0, i, 0)),
            pl.BlockSpec((D,), lambda i: (0,)),
            pl.BlockSpec((D, 1), lambda i: (0, 0)),
            pl.BlockSpec((1,), lambda i: (0,)),
        ],
        out_specs=pl.BlockSpec((1, 1), lambda i: (0, 0)),
        out_shape=jax.ShapeDtypeStruct((1, 1), jnp.float32),
    )(aggp, deginp, b3, Wl, bl)


def kernel(x, edge_index, W1, b1, W2, b2, W3, b3, Wl, bl):
    pad = N + (jnp.arange(EP - E, dtype=edge_index.dtype) % (NP - N))
    src = jnp.concatenate([edge_index[0], pad]).reshape(NW * KPT, IW)
    dst = jnp.concatenate([edge_index[1], pad]).reshape(NW * KPT, IW)
    xp = jnp.pad(x, ((0, NP - N), (0, 0)))

    degp = _sc_degrees(src, dst)
    srcf = edge_index[0]
    dstf = edge_index[1]
    degsrc = degp[:, 0, :].reshape(NC, NP, 1)
    degdst = degp[:, 1, :].reshape(NC, NP, 1)

    hw1 = _tc_first(xp, degsrc, W1)
    agg1 = _sc_scatter(hw1, srcf, dstf)
    hw2 = _tc_mid(agg1, degdst, degsrc, b1, W2)
    agg2 = _sc_scatter(hw2, srcf, dstf)
    hw3 = _tc_mid(agg2, degdst, degsrc, b2, W3)
    agg3 = _sc_scatter(hw3, srcf, dstf)
    return _tc_final(agg3, degdst, b3, Wl, bl)
